# baseline (speedup 1.0000x reference)
import functools

import jax
import jax.numpy as jnp
from jax import lax
from jax.experimental import pallas as pl
from jax.experimental.pallas import tpu as pltpu
from jax.experimental.pallas import tpu_sc as plsc

F32 = jnp.float32

TAB_W = 640
QK_W = 128
MSG_TAB_W = 512
N_HEADS = 4
D_QK = 16
D_HEAD = 120
D_NODE = 480
D_EDGE = 184
N_BASIS = 128
D_SCALAR = 128

NODE_BLK = 1000
EDGE_BLK = 640
MSG_W = 128


def _ln(x, eps=1e-6):
    m = jnp.mean(x, axis=-1, keepdims=True)
    d = x - m
    v = jnp.mean(d * d, axis=-1, keepdims=True)
    return d / jnp.sqrt(v + eps)


def _silu(x):
    return x * jax.nn.sigmoid(x)


def _node_stage_body(natom_ref, pos_ref, atab_ref,
                     qW1_ref, qb1_ref, qg1_ref, qbe1_ref, qW2_ref, qb2_ref,
                     kW1_ref, kb1_ref, kg1_ref, kbe1_ref, kW2_ref, kb2_ref,
                     preWs_ref, preWd_ref, skipW_ref,
                     sq_ref, dq_ref, sm_ref, dm_ref, skip_ref):
    atom = natom_ref[0, 0, :]
    oh = (atom.reshape(NODE_BLK, 1) ==
          lax.broadcasted_iota(jnp.int32, (1, 128), 1)).astype(F32)
    node_fea = jnp.dot(oh, atab_ref[...], preferred_element_type=F32)
    nf = _ln(node_fea)
    ns = nf[:, :D_SCALAR]

    def mlp(W1, b1, g1, be1, W2, b2):
        h = jnp.dot(ns, W1[...], preferred_element_type=F32) + b1[...]
        h = _silu(_ln(h) * g1[...] + be1[...])
        return jnp.dot(h, W2[...], preferred_element_type=F32) + b2[...]

    q_node = mlp(qW1_ref, qb1_ref, qg1_ref, qbe1_ref, qW2_ref, qb2_ref)
    k_node = mlp(kW1_ref, kb1_ref, kg1_ref, kbe1_ref, kW2_ref, kb2_ref)
    msg_s = jnp.dot(nf, preWs_ref[...], preferred_element_type=F32)
    msg_d = jnp.dot(nf, preWd_ref[...], preferred_element_type=F32)
    pos = pos_ref[...]
    zpad = jnp.zeros((NODE_BLK, 48), F32)
    zpad2 = jnp.zeros((NODE_BLK, 32), jnp.bfloat16)
    sq_ref[...] = jnp.concatenate([k_node, pos, zpad], axis=1)
    dq_ref[...] = jnp.concatenate([q_node, pos, zpad], axis=1)
    sm_ref[...] = jnp.concatenate([msg_s.astype(jnp.bfloat16), zpad2], axis=1)
    dm_ref[...] = jnp.concatenate([msg_d.astype(jnp.bfloat16), zpad2], axis=1)
    skip_ref[...] = jnp.dot(node_fea, skipW_ref[...],
                            preferred_element_type=F32)


def _node_stage(node_atom, pos, p, n_nodes, interpret=False):
    nblk = n_nodes // NODE_BLK
    natom3 = node_atom.astype(jnp.int32).reshape(nblk, 1, NODE_BLK)
    pos16 = jnp.pad(pos.astype(F32), ((0, 0), (0, 13)))

    def full(a):
        return pl.BlockSpec(a.shape, lambda i: (0,) * a.ndim)

    r1 = lambda v: v.reshape(1, -1)
    weights = [p['atom_table'],
               p['q_W1'], r1(p['q_b1']), r1(p['q_g1']), r1(p['q_be1']),
               p['q_W2'], r1(p['q_b2']),
               p['k_W1'], r1(p['k_b1']), r1(p['k_g1']), r1(p['k_be1']),
               p['k_W2'], r1(p['k_b2']),
               p['pre_Ws'], p['pre_Wd'], p['skip_n_W']]
    out = pl.pallas_call(
        _node_stage_body,
        grid=(nblk,),
        in_specs=[pl.BlockSpec((1, 1, NODE_BLK), lambda i: (i, 0, 0)),
                  pl.BlockSpec((NODE_BLK, 16), lambda i: (i, 0))] +
                 [full(w) for w in weights],
        out_specs=[pl.BlockSpec((NODE_BLK, QK_W), lambda i: (i, 0)),
                   pl.BlockSpec((NODE_BLK, QK_W), lambda i: (i, 0)),
                   pl.BlockSpec((NODE_BLK, MSG_TAB_W), lambda i: (i, 0)),
                   pl.BlockSpec((NODE_BLK, MSG_TAB_W), lambda i: (i, 0)),
                   pl.BlockSpec((NODE_BLK, D_NODE), lambda i: (i, 0))],
        out_shape=[jax.ShapeDtypeStruct((n_nodes, QK_W), F32),
                   jax.ShapeDtypeStruct((n_nodes, QK_W), F32),
                   jax.ShapeDtypeStruct((n_nodes, MSG_TAB_W), jnp.bfloat16),
                   jax.ShapeDtypeStruct((n_nodes, MSG_TAB_W), jnp.bfloat16),
                   jax.ShapeDtypeStruct((n_nodes, D_NODE), F32)],
        interpret=interpret,
    )(natom3, pos16, *weights)
    return out


def _edge_stage_body(sq_ref, sm_ref, dq_ref, dm_ref,
                     Wep_ref, bep_ref, shc_ref,
                     aW1_ref, ab1_ref, ag1_ref, abe1_ref,
                     aW2_ref, ab2_ref, ag2_ref, abe2_ref,
                     aW3_ref, ab3_ref,
                     preWe_ref, preb_ref,
                     c1W_ref, c1Wr_ref, c1b_ref,
                     c2W_ref, c2Wr_ref, c2b_ref,
                     onW_ref, lineW_ref, lineb_ref, skipeW_ref,
                     eout_ref, msg4_ref):
    s_qk = sq_ref[...]
    d_qk = dq_ref[...]
    B = EDGE_BLK

    ev = d_qk[:, 64:80] - s_qk[:, 64:80]
    ev2 = ev * ev
    r2_1 = jnp.sum(ev2, axis=1, keepdims=True)
    rb = jnp.sqrt(jnp.broadcast_to(r2_1, (B, N_BASIS)))
    invr = 1.0 / jnp.maximum(rb, 1e-12)
    X = jnp.broadcast_to(ev[:, 0:1], (B, N_BASIS)) * invr
    Y = jnp.broadcast_to(ev[:, 1:2], (B, N_BASIS)) * invr
    Z = jnp.broadcast_to(ev[:, 2:3], (B, N_BASIS)) * invr

    shc = shc_ref[...]
    C1, CX, CY, CZ = shc[0:1], shc[1:2], shc[2:3], shc[3:4]
    CXY, CYZ, CXZ = shc[4:5], shc[5:6], shc[6:7]
    CXX, CYY, CZZ = shc[7:8], shc[8:9], shc[9:10]
    sh128 = (C1 +
             X * (CX + CXY * Y + CXZ * Z + CXX * X) +
             Y * (CY + CYZ * Z + CYY * Y) +
             Z * (CZ + CZZ * Z))
    edge_fea = (jnp.dot(sh128, Wep_ref[...], preferred_element_type=F32) +
                bep_ref[...])
    ef = _ln(edge_fea)

    centers = (lax.broadcasted_iota(jnp.int32, (1, N_BASIS), 1).astype(F32)
               * (10.0 / (N_BASIS - 1)))
    width = 0.5 * 10.0 / N_BASIS
    dr = rb - centers
    rbf = jnp.exp(-(dr * dr) / (2.0 * width * width))

    a = jnp.dot(rbf, aW1_ref[...], preferred_element_type=F32) + ab1_ref[...]
    a = _silu(_ln(a) * ag1_ref[...] + abe1_ref[...])
    a = jnp.dot(a, aW2_ref[...], preferred_element_type=F32) + ab2_ref[...]
    a = _silu(_ln(a) * ag2_ref[...] + abe2_ref[...])
    edge_bias = jnp.dot(a, aW3_ref[...], preferred_element_type=F32) + ab3_ref[...]

    qk = d_qk[:, 0:64] * s_qk[:, 0:64]
    qk_h = jnp.concatenate(
        [jnp.sum(qk[:, h * D_QK:(h + 1) * D_QK], axis=1, keepdims=True)
         for h in range(N_HEADS)], axis=1)
    alpha = qk_h * (1.0 / (D_QK ** 0.5)) + edge_bias

    msg = ((sm_ref[:, 0:D_NODE].astype(F32) +
            dm_ref[:, 0:D_NODE].astype(F32)) +
           jnp.dot(ef, preWe_ref[...], preferred_element_type=F32) +
           preb_ref[...])
    t = (jnp.dot(msg, c1W_ref[...], preferred_element_type=F32) +
         jnp.dot(rbf, c1Wr_ref[...], preferred_element_type=F32) + c1b_ref[...])
    v = _silu(_ln(t))
    v = (jnp.dot(v, c2W_ref[...], preferred_element_type=F32) +
         jnp.dot(rbf, c2Wr_ref[...], preferred_element_type=F32) + c2b_ref[...])
    von = jnp.dot(msg, onW_ref[...], preferred_element_type=F32)
    r_480 = jnp.sqrt(jnp.broadcast_to(r2_1, (B, D_NODE)))
    v = jnp.where(r_480 < 1e-10, von, v)

    em = jnp.concatenate(
        [v[:, h * D_HEAD:(h + 1) * D_HEAD] * alpha[:, h:h + 1]
         for h in range(N_HEADS)], axis=1)

    eout_ref[...] = (jnp.dot(em, lineW_ref[...], preferred_element_type=F32) +
                     lineb_ref[...] +
                     jnp.dot(edge_fea, skipeW_ref[...],
                             preferred_element_type=F32))
    zpad8 = jnp.zeros((B, MSG_W - D_HEAD), F32)
    for h in range(N_HEADS):
        msg4_ref[h, :, :] = jnp.concatenate(
            [em[:, h * D_HEAD:(h + 1) * D_HEAD], zpad8], axis=1)


def _edge_stage(s_qk, s_msg, d_qk, d_msg, p, n_edges, interpret=False):
    eblk = n_edges // EDGE_BLK

    def full(a):
        return pl.BlockSpec(a.shape, lambda i: (0,) * a.ndim)

    r1 = lambda v: v.reshape(1, -1)
    s3, s5, s15 = 3.0 ** 0.5, 5.0 ** 0.5, 15.0 ** 0.5
    shc = jnp.zeros((10, 16), F32)
    shc = shc.at[0, 0].set(1.0).at[0, 6].set(-s5 / 2.0)
    shc = shc.at[1, 1].set(s3).at[2, 2].set(s3).at[3, 3].set(s3)
    shc = shc.at[4, 4].set(s15).at[5, 5].set(s15).at[6, 7].set(s15)
    shc = shc.at[7, 8].set(s15 / 2.0).at[8, 8].set(-s15 / 2.0)
    shc = shc.at[9, 6].set(3.0 * s5 / 2.0)
    shc = jnp.pad(shc, ((0, 0), (0, N_BASIS - 16)))
    wep128 = jnp.pad(p['W_edge_pre'], ((0, N_BASIS - 9), (0, 0)))
    weights = [wep128, r1(p['b_edge_pre']), shc,
               p['a_W1'], r1(p['a_b1']), r1(p['a_g1']), r1(p['a_be1']),
               p['a_W2'], r1(p['a_b2']), r1(p['a_g2']), r1(p['a_be2']),
               p['a_W3'], r1(p['a_b3']),
               p['pre_We'], r1(p['pre_b']),
               p['c1_W'], p['c1_Wr'], r1(p['c1_b']),
               p['c2_W'], p['c2_Wr'], r1(p['c2_b']),
               p['on_W'], p['lin_e_W'], r1(p['lin_e_b']), p['skip_e_W']]
    return pl.pallas_call(
        _edge_stage_body,
        grid=(eblk,),
        in_specs=[pl.BlockSpec((EDGE_BLK, QK_W), lambda i: (i, 0)),
                  pl.BlockSpec((EDGE_BLK, MSG_TAB_W), lambda i: (i, 0)),
                  pl.BlockSpec((EDGE_BLK, QK_W), lambda i: (i, 0)),
                  pl.BlockSpec((EDGE_BLK, MSG_TAB_W), lambda i: (i, 0))] +
                 [full(w) for w in weights],
        out_specs=[pl.BlockSpec((EDGE_BLK, D_EDGE), lambda i: (i, 0)),
                   pl.BlockSpec((N_HEADS, EDGE_BLK, MSG_W),
                                lambda i: (0, i, 0))],
        out_shape=[jax.ShapeDtypeStruct((n_edges, D_EDGE), F32),
                   jax.ShapeDtypeStruct((N_HEADS, n_edges, MSG_W), F32)],
        interpret=interpret,
    )(s_qk, s_msg, d_qk, d_msg, *weights)


def _node_out_body(*refs):
    n_parts = len(refs) - 4
    part_refs = refs[:n_parts]
    skip_ref, linW_ref, linb_ref, out_ref = refs[n_parts:]
    m4 = part_refs[0][...]
    for pr in part_refs[1:]:
        m4 = m4 + pr[...]
    acc = skip_ref[...] + linb_ref[...]
    for h in range(N_HEADS):
        acc = acc + jnp.dot(m4[h], linW_ref[h, :, :],
                            preferred_element_type=F32)
    out_ref[...] = acc


def _node_out_stage(msg4n_parts, skip_n, p, n_nodes, interpret=False):
    nblk = n_nodes // NODE_BLK
    linW4 = jnp.pad(p['lin_W'].reshape(N_HEADS, D_HEAD, D_NODE),
                    ((0, 0), (0, MSG_W - D_HEAD), (0, 0)))
    linb = p['lin_b'].reshape(1, -1)
    part_spec = pl.BlockSpec((N_HEADS, NODE_BLK, MSG_W), lambda i: (0, i, 0))
    return pl.pallas_call(
        _node_out_body,
        grid=(nblk,),
        in_specs=[part_spec] * len(msg4n_parts) +
                 [pl.BlockSpec((NODE_BLK, D_NODE), lambda i: (i, 0)),
                  pl.BlockSpec(linW4.shape, lambda i: (0, 0, 0)),
                  pl.BlockSpec(linb.shape, lambda i: (0, 0))],
        out_specs=pl.BlockSpec((NODE_BLK, D_NODE), lambda i: (i, 0)),
        out_shape=jax.ShapeDtypeStruct((n_nodes, D_NODE), F32),
        interpret=interpret,
    )(*msg4n_parts, skip_n, linW4, linb)


def _sc_gather(sq_tab, sm_tab, dq_tab, dm_tab, esrc, edst, n_edges):
    info = plsc.get_sparse_core_info()
    nw = info.num_cores * info.num_subcores
    per_w = n_edges // nw
    CH = 40
    n_ch = per_w // CH
    MW = MSG_TAB_W // 2
    mesh = plsc.VectorSubcoreMesh(core_axis_name="c", subcore_axis_name="s")

    @functools.partial(
        pl.kernel, mesh=mesh,
        out_type=[jax.ShapeDtypeStruct((n_edges, QK_W), F32),
                  jax.ShapeDtypeStruct((n_edges, MW), F32),
                  jax.ShapeDtypeStruct((n_edges, QK_W), F32),
                  jax.ShapeDtypeStruct((n_edges, MW), F32)],
        scratch_types=[
            pltpu.VMEM((CH,), jnp.int32),
            pltpu.VMEM((CH,), jnp.int32),
            pltpu.VMEM((CH, QK_W), F32),
            pltpu.VMEM((CH, MW), F32),
            pltpu.VMEM((CH, QK_W), F32),
            pltpu.VMEM((CH, MW), F32),
            pltpu.SemaphoreType.DMA,
            pltpu.SemaphoreType.DMA,
            pltpu.SemaphoreType.DMA,
            pltpu.SemaphoreType.DMA,
        ],
    )
    def gkern(sq_hbm, sm_hbm, dq_hbm, dm_hbm, esrc_hbm, edst_hbm,
              sqo_hbm, smo_hbm, dqo_hbm, dmo_hbm,
              idx_s, idx_d, r_sq, r_sm, r_dq, r_dm,
              sem1, sem2, sem3, sem4):
        wid = lax.axis_index("s") * info.num_cores + lax.axis_index("c")
        base = wid * per_w

        def step(j, _):
            off = base + j * CH
            pltpu.sync_copy(esrc_hbm.at[pl.ds(off, CH)], idx_s)
            pltpu.sync_copy(edst_hbm.at[pl.ds(off, CH)], idx_d)
            c1 = pltpu.async_copy(sq_hbm.at[idx_s], r_sq, sem1)
            c3 = pltpu.async_copy(dq_hbm.at[idx_d], r_dq, sem3)
            c1.wait()
            c3.wait()
            c2 = pltpu.async_copy(sm_hbm.at[idx_s], r_sm, sem2)
            c4 = pltpu.async_copy(dm_hbm.at[idx_d], r_dm, sem4)
            pltpu.sync_copy(r_sq, sqo_hbm.at[pl.ds(off, CH)])
            pltpu.sync_copy(r_dq, dqo_hbm.at[pl.ds(off, CH)])
            c2.wait()
            c4.wait()
            pltpu.sync_copy(r_sm, smo_hbm.at[pl.ds(off, CH)])
            pltpu.sync_copy(r_dm, dmo_hbm.at[pl.ds(off, CH)])
            return ()

        lax.fori_loop(0, n_ch, step, ())

    return gkern(sq_tab, sm_tab, dq_tab, dm_tab, esrc, edst)


def _sc_scatter(msg4, edst, n_nodes, n_edges):
    info = plsc.get_sparse_core_info()
    ns = info.num_subcores
    per_s = n_edges // ns
    CH = 80
    n_ch = per_s // CH
    n_pad = ((n_nodes + 8 * ns - 1) // (8 * ns)) * (8 * ns)
    rows_per_s = n_pad // ns
    ZCH = 128
    mesh = plsc.VectorSubcoreMesh(core_axis_name="c", subcore_axis_name="s")

    @functools.partial(
        pl.kernel, mesh=mesh,
        out_type=jax.ShapeDtypeStruct((N_HEADS, n_pad, MSG_W), F32),
        scratch_types=[
            pltpu.VMEM((CH, MSG_W), F32),
            pltpu.VMEM((CH,), jnp.int32),
            pltpu.VMEM((ZCH, MSG_W), F32),
            pltpu.VMEM_SHARED((n_pad, MSG_W), F32),
        ],
    )
    def skern(msg4_hbm, edst_hbm, out_hbm, buf, idxb, zbuf, acc):
        core = lax.axis_index("c")
        sid = lax.axis_index("s")

        zbuf[...] = jnp.zeros((ZCH, MSG_W), F32)

        def one_head(h):
            for zz in range(rows_per_s // ZCH):
                pltpu.sync_copy(
                    zbuf, acc.at[pl.ds(sid * rows_per_s + zz * ZCH, ZCH)])
            plsc.subcore_barrier()

            def step(j, _):
                off = sid * per_s + j * CH
                pltpu.sync_copy(edst_hbm.at[pl.ds(off, CH)], idxb)
                pltpu.sync_copy(msg4_hbm.at[h, pl.ds(off, CH)], buf)
                pltpu.sync_copy(buf, acc.at[idxb], add=True)
                return ()

            lax.fori_loop(0, n_ch, step, ())
            plsc.subcore_barrier()
            pltpu.sync_copy(
                acc.at[pl.ds(sid * rows_per_s, rows_per_s)],
                out_hbm.at[h, pl.ds(sid * rows_per_s, rows_per_s)])
            plsc.subcore_barrier()

        one_head(core * 2)
        one_head(core * 2 + 1)

    return skern(msg4, edst)


def kernel(node_atom, pos, edge_src, edge_dst, batch, params):
    n_nodes = node_atom.shape[0]
    n_edges = edge_src.shape[0]
    esrc = edge_src.astype(jnp.int32)
    edst = edge_dst.astype(jnp.int32)
    sq_tab, dq_tab, sm_tab, dm_tab, skip_n = _node_stage(
        node_atom, pos, params, n_nodes)
    MW = MSG_TAB_W // 2
    as_f32 = lambda a: lax.bitcast_convert_type(
        a.reshape(n_nodes, MW, 2), F32)
    sm32, dm32 = as_f32(sm_tab), as_f32(dm_tab)
    n_sl = 5
    e_sl = n_edges // n_sl
    eouts, parts = [], []
    for i in range(n_sl):
        lo = i * e_sl
        sq_g, sm_g, dq_g, dm_g = _sc_gather(
            sq_tab, sm32, dq_tab, dm32,
            lax.dynamic_slice(esrc, (lo,), (e_sl,)),
            lax.dynamic_slice(edst, (lo,), (e_sl,)), e_sl)
        as_bf = lambda a: lax.bitcast_convert_type(
            a, jnp.bfloat16).reshape(e_sl, MSG_TAB_W)
        eo, msg4 = _edge_stage(sq_g, as_bf(sm_g), dq_g, as_bf(dm_g),
                               params, e_sl)
        eouts.append(eo)
        parts.append(_sc_scatter(msg4,
                                 lax.dynamic_slice(edst, (lo,), (e_sl,)),
                                 n_nodes, e_sl))
    edge_out = jnp.concatenate(eouts, axis=0)
    node_out = _node_out_stage(parts, skip_n, params, n_nodes)
    return node_out, edge_out


# speedup vs baseline: 3.1397x; 3.1397x over previous
import functools

import jax
import jax.numpy as jnp
from jax import lax
from jax.experimental import pallas as pl
from jax.experimental.pallas import tpu as pltpu
from jax.experimental.pallas import tpu_sc as plsc

F32 = jnp.float32

TAB_W = 640
QK_W = 128
MSG_TAB_W = 512
MSG_PACK_W = 256
N_HEADS = 4
D_QK = 16
D_HEAD = 120
D_NODE = 480
D_EDGE = 184
N_BASIS = 128
D_SCALAR = 128

NODE_BLK = 1000
EDGE_BLK = 640
MSG_W = 128


def _ln(x, eps=1e-6):
    m = jnp.mean(x, axis=-1, keepdims=True)
    d = x - m
    v = jnp.mean(d * d, axis=-1, keepdims=True)
    return d / jnp.sqrt(v + eps)


def _silu(x):
    return x * jax.nn.sigmoid(x)


def _node_stage_body(natom_ref, pos_ref, atab_ref,
                     qW1_ref, qb1_ref, qg1_ref, qbe1_ref, qW2_ref, qb2_ref,
                     kW1_ref, kb1_ref, kg1_ref, kbe1_ref, kW2_ref, kb2_ref,
                     preWs_ref, preWd_ref, skipW_ref,
                     sq_ref, dq_ref, sm_ref, dm_ref, skip_ref):
    atom = natom_ref[0, 0, :]
    oh = (atom.reshape(NODE_BLK, 1) ==
          lax.broadcasted_iota(jnp.int32, (1, 128), 1)).astype(F32)
    node_fea = jnp.dot(oh, atab_ref[...], preferred_element_type=F32)
    nf = _ln(node_fea)
    ns = nf[:, :D_SCALAR]

    def mlp(W1, b1, g1, be1, W2, b2):
        h = jnp.dot(ns, W1[...], preferred_element_type=F32) + b1[...]
        h = _silu(_ln(h) * g1[...] + be1[...])
        return jnp.dot(h, W2[...], preferred_element_type=F32) + b2[...]

    q_node = mlp(qW1_ref, qb1_ref, qg1_ref, qbe1_ref, qW2_ref, qb2_ref)
    k_node = mlp(kW1_ref, kb1_ref, kg1_ref, kbe1_ref, kW2_ref, kb2_ref)
    msg_s = jnp.dot(nf, preWs_ref[...], preferred_element_type=F32)
    msg_d = jnp.dot(nf, preWd_ref[...], preferred_element_type=F32)
    pos = pos_ref[...]
    zpad = jnp.zeros((NODE_BLK, 48), F32)
    sq_ref[...] = jnp.concatenate([k_node, pos, zpad], axis=1)
    dq_ref[...] = jnp.concatenate([q_node, pos, zpad], axis=1)

    def pack_pair(m):
        lo = m[:, 0:256].astype(jnp.bfloat16).astype(F32)
        hi = jnp.concatenate(
            [m[:, 256:480], jnp.zeros((NODE_BLK, 32), F32)],
            axis=1).astype(jnp.bfloat16).astype(F32)
        ulo = lax.bitcast_convert_type(lo, jnp.uint32) >> 16
        uhi = lax.bitcast_convert_type(hi, jnp.uint32) & jnp.uint32(0xFFFF0000)
        return lax.bitcast_convert_type(ulo | uhi, F32)

    sm_ref[...] = pack_pair(msg_s)
    dm_ref[...] = pack_pair(msg_d)
    skip_ref[...] = jnp.dot(node_fea, skipW_ref[...],
                            preferred_element_type=F32)


def _node_stage(node_atom, pos, p, n_nodes, interpret=False):
    nblk = n_nodes // NODE_BLK
    natom3 = node_atom.astype(jnp.int32).reshape(nblk, 1, NODE_BLK)
    pos16 = jnp.pad(pos.astype(F32), ((0, 0), (0, 13)))

    def full(a):
        return pl.BlockSpec(a.shape, lambda i: (0,) * a.ndim)

    r1 = lambda v: v.reshape(1, -1)
    weights = [p['atom_table'],
               p['q_W1'], r1(p['q_b1']), r1(p['q_g1']), r1(p['q_be1']),
               p['q_W2'], r1(p['q_b2']),
               p['k_W1'], r1(p['k_b1']), r1(p['k_g1']), r1(p['k_be1']),
               p['k_W2'], r1(p['k_b2']),
               p['pre_Ws'], p['pre_Wd'], p['skip_n_W']]
    out = pl.pallas_call(
        _node_stage_body,
        grid=(nblk,),
        in_specs=[pl.BlockSpec((1, 1, NODE_BLK), lambda i: (i, 0, 0)),
                  pl.BlockSpec((NODE_BLK, 16), lambda i: (i, 0))] +
                 [full(w) for w in weights],
        out_specs=[pl.BlockSpec((NODE_BLK, QK_W), lambda i: (i, 0)),
                   pl.BlockSpec((NODE_BLK, QK_W), lambda i: (i, 0)),
                   pl.BlockSpec((NODE_BLK, MSG_PACK_W), lambda i: (i, 0)),
                   pl.BlockSpec((NODE_BLK, MSG_PACK_W), lambda i: (i, 0)),
                   pl.BlockSpec((NODE_BLK, D_NODE), lambda i: (i, 0))],
        out_shape=[jax.ShapeDtypeStruct((n_nodes, QK_W), F32),
                   jax.ShapeDtypeStruct((n_nodes, QK_W), F32),
                   jax.ShapeDtypeStruct((n_nodes, MSG_PACK_W), F32),
                   jax.ShapeDtypeStruct((n_nodes, MSG_PACK_W), F32),
                   jax.ShapeDtypeStruct((n_nodes, D_NODE), F32)],
        interpret=interpret,
    )(natom3, pos16, *weights)
    return out


def _edge_stage_body(sq_ref, sm_ref, dq_ref, dm_ref,
                     Wep_ref, bep_ref, shc_ref,
                     aW1_ref, ab1_ref, ag1_ref, abe1_ref,
                     aW2_ref, ab2_ref, ag2_ref, abe2_ref,
                     aW3_ref, ab3_ref,
                     preWe_ref, preb_ref,
                     c1W_ref, c1Wr_ref, c1b_ref,
                     c2W_ref, c2Wr_ref, c2b_ref,
                     onW_ref, lineW_ref, lineb_ref, skipeW_ref,
                     eout_ref, msg4_ref):
    s_qk = sq_ref[...]
    d_qk = dq_ref[...]
    B = EDGE_BLK

    ev = d_qk[:, 64:80] - s_qk[:, 64:80]
    ev2 = ev * ev
    r2_1 = jnp.sum(ev2, axis=1, keepdims=True)
    rb = jnp.sqrt(jnp.broadcast_to(r2_1, (B, N_BASIS)))
    invr = 1.0 / jnp.maximum(rb, 1e-12)
    X = jnp.broadcast_to(ev[:, 0:1], (B, N_BASIS)) * invr
    Y = jnp.broadcast_to(ev[:, 1:2], (B, N_BASIS)) * invr
    Z = jnp.broadcast_to(ev[:, 2:3], (B, N_BASIS)) * invr

    shc = shc_ref[...]
    C1, CX, CY, CZ = shc[0:1], shc[1:2], shc[2:3], shc[3:4]
    CXY, CYZ, CXZ = shc[4:5], shc[5:6], shc[6:7]
    CXX, CYY, CZZ = shc[7:8], shc[8:9], shc[9:10]
    sh128 = (C1 +
             X * (CX + CXY * Y + CXZ * Z + CXX * X) +
             Y * (CY + CYZ * Z + CYY * Y) +
             Z * (CZ + CZZ * Z))
    edge_fea = (jnp.dot(sh128, Wep_ref[...], preferred_element_type=F32) +
                bep_ref[...])
    ef = _ln(edge_fea)

    centers = (lax.broadcasted_iota(jnp.int32, (1, N_BASIS), 1).astype(F32)
               * (10.0 / (N_BASIS - 1)))
    width = 0.5 * 10.0 / N_BASIS
    dr = rb - centers
    rbf = jnp.exp(-(dr * dr) / (2.0 * width * width))

    a = jnp.dot(rbf, aW1_ref[...], preferred_element_type=F32) + ab1_ref[...]
    a = _silu(_ln(a) * ag1_ref[...] + abe1_ref[...])
    a = jnp.dot(a, aW2_ref[...], preferred_element_type=F32) + ab2_ref[...]
    a = _silu(_ln(a) * ag2_ref[...] + abe2_ref[...])
    edge_bias = jnp.dot(a, aW3_ref[...], preferred_element_type=F32) + ab3_ref[...]

    qk = d_qk[:, 0:64] * s_qk[:, 0:64]
    qk_h = jnp.concatenate(
        [jnp.sum(qk[:, h * D_QK:(h + 1) * D_QK], axis=1, keepdims=True)
         for h in range(N_HEADS)], axis=1)
    alpha = qk_h * (1.0 / (D_QK ** 0.5)) + edge_bias

    def unpack_pair(ref):
        u = lax.bitcast_convert_type(ref[...], jnp.uint32)
        lo = lax.bitcast_convert_type(u << 16, F32)
        hi = lax.bitcast_convert_type(u & jnp.uint32(0xFFFF0000), F32)
        return jnp.concatenate([lo, hi[:, 0:D_NODE - MSG_PACK_W]], axis=1)

    msg = ((unpack_pair(sm_ref) + unpack_pair(dm_ref)) +
           jnp.dot(ef, preWe_ref[...], preferred_element_type=F32) +
           preb_ref[...])
    t = (jnp.dot(msg, c1W_ref[...], preferred_element_type=F32) +
         jnp.dot(rbf, c1Wr_ref[...], preferred_element_type=F32) + c1b_ref[...])
    v = _silu(_ln(t))
    v = (jnp.dot(v, c2W_ref[...], preferred_element_type=F32) +
         jnp.dot(rbf, c2Wr_ref[...], preferred_element_type=F32) + c2b_ref[...])
    von = jnp.dot(msg, onW_ref[...], preferred_element_type=F32)
    r_480 = jnp.sqrt(jnp.broadcast_to(r2_1, (B, D_NODE)))
    v = jnp.where(r_480 < 1e-10, von, v)

    em = jnp.concatenate(
        [v[:, h * D_HEAD:(h + 1) * D_HEAD] * alpha[:, h:h + 1]
         for h in range(N_HEADS)], axis=1)

    eout_ref[...] = (jnp.dot(em, lineW_ref[...], preferred_element_type=F32) +
                     lineb_ref[...] +
                     jnp.dot(edge_fea, skipeW_ref[...],
                             preferred_element_type=F32))
    zpad8 = jnp.zeros((B, MSG_W - D_HEAD), F32)
    for h in range(N_HEADS):
        msg4_ref[h, :, :] = jnp.concatenate(
            [em[:, h * D_HEAD:(h + 1) * D_HEAD], zpad8], axis=1)


def _edge_stage(s_qk, s_msg, d_qk, d_msg, p, n_edges, interpret=False):
    eblk = n_edges // EDGE_BLK

    def full(a):
        return pl.BlockSpec(a.shape, lambda i: (0,) * a.ndim)

    r1 = lambda v: v.reshape(1, -1)
    s3, s5, s15 = 3.0 ** 0.5, 5.0 ** 0.5, 15.0 ** 0.5
    shc = jnp.zeros((10, 16), F32)
    shc = shc.at[0, 0].set(1.0).at[0, 6].set(-s5 / 2.0)
    shc = shc.at[1, 1].set(s3).at[2, 2].set(s3).at[3, 3].set(s3)
    shc = shc.at[4, 4].set(s15).at[5, 5].set(s15).at[6, 7].set(s15)
    shc = shc.at[7, 8].set(s15 / 2.0).at[8, 8].set(-s15 / 2.0)
    shc = shc.at[9, 6].set(3.0 * s5 / 2.0)
    shc = jnp.pad(shc, ((0, 0), (0, N_BASIS - 16)))
    wep128 = jnp.pad(p['W_edge_pre'], ((0, N_BASIS - 9), (0, 0)))
    weights = [wep128, r1(p['b_edge_pre']), shc,
               p['a_W1'], r1(p['a_b1']), r1(p['a_g1']), r1(p['a_be1']),
               p['a_W2'], r1(p['a_b2']), r1(p['a_g2']), r1(p['a_be2']),
               p['a_W3'], r1(p['a_b3']),
               p['pre_We'], r1(p['pre_b']),
               p['c1_W'], p['c1_Wr'], r1(p['c1_b']),
               p['c2_W'], p['c2_Wr'], r1(p['c2_b']),
               p['on_W'], p['lin_e_W'], r1(p['lin_e_b']), p['skip_e_W']]
    return pl.pallas_call(
        _edge_stage_body,
        grid=(eblk,),
        in_specs=[pl.BlockSpec((EDGE_BLK, QK_W), lambda i: (i, 0)),
                  pl.BlockSpec((EDGE_BLK, MSG_PACK_W), lambda i: (i, 0)),
                  pl.BlockSpec((EDGE_BLK, QK_W), lambda i: (i, 0)),
                  pl.BlockSpec((EDGE_BLK, MSG_PACK_W), lambda i: (i, 0))] +
                 [full(w) for w in weights],
        out_specs=[pl.BlockSpec((EDGE_BLK, D_EDGE), lambda i: (i, 0)),
                   pl.BlockSpec((N_HEADS, EDGE_BLK, MSG_W),
                                lambda i: (0, i, 0))],
        out_shape=[jax.ShapeDtypeStruct((n_edges, D_EDGE), F32),
                   jax.ShapeDtypeStruct((N_HEADS, n_edges, MSG_W), F32)],
        interpret=interpret,
    )(s_qk, s_msg, d_qk, d_msg, *weights)


def _node_out_body(*refs):
    n_parts = len(refs) - 4
    part_refs = refs[:n_parts]
    skip_ref, linW_ref, linb_ref, out_ref = refs[n_parts:]
    m4 = part_refs[0][...]
    for pr in part_refs[1:]:
        m4 = m4 + pr[...]
    acc = skip_ref[...] + linb_ref[...]
    for h in range(N_HEADS):
        acc = acc + jnp.dot(m4[h], linW_ref[h, :, :],
                            preferred_element_type=F32)
    out_ref[...] = acc


def _node_out_stage(msg4n_parts, skip_n, p, n_nodes, interpret=False):
    nblk = n_nodes // NODE_BLK
    linW4 = jnp.pad(p['lin_W'].reshape(N_HEADS, D_HEAD, D_NODE),
                    ((0, 0), (0, MSG_W - D_HEAD), (0, 0)))
    linb = p['lin_b'].reshape(1, -1)
    part_spec = pl.BlockSpec((N_HEADS, NODE_BLK, MSG_W), lambda i: (0, i, 0))
    return pl.pallas_call(
        _node_out_body,
        grid=(nblk,),
        in_specs=[part_spec] * len(msg4n_parts) +
                 [pl.BlockSpec((NODE_BLK, D_NODE), lambda i: (i, 0)),
                  pl.BlockSpec(linW4.shape, lambda i: (0, 0, 0)),
                  pl.BlockSpec(linb.shape, lambda i: (0, 0))],
        out_specs=pl.BlockSpec((NODE_BLK, D_NODE), lambda i: (i, 0)),
        out_shape=jax.ShapeDtypeStruct((n_nodes, D_NODE), F32),
        interpret=interpret,
    )(*msg4n_parts, skip_n, linW4, linb)


def _sc_gather(sq_tab, sm_tab, dq_tab, dm_tab, esrc, edst, n_edges):
    info = plsc.get_sparse_core_info()
    nw = info.num_cores * info.num_subcores
    per_w = n_edges // nw
    CH = 40
    n_ch = per_w // CH
    MW = MSG_PACK_W
    mesh = plsc.VectorSubcoreMesh(core_axis_name="c", subcore_axis_name="s")

    @functools.partial(
        pl.kernel, mesh=mesh,
        out_type=[jax.ShapeDtypeStruct((n_edges, QK_W), F32),
                  jax.ShapeDtypeStruct((n_edges, MW), F32),
                  jax.ShapeDtypeStruct((n_edges, QK_W), F32),
                  jax.ShapeDtypeStruct((n_edges, MW), F32)],
        scratch_types=[
            pltpu.VMEM((CH,), jnp.int32),
            pltpu.VMEM((CH,), jnp.int32),
            pltpu.VMEM((CH, QK_W), F32),
            pltpu.VMEM((CH, MW), F32),
            pltpu.VMEM((CH, QK_W), F32),
            pltpu.VMEM((CH, MW), F32),
            pltpu.SemaphoreType.DMA,
            pltpu.SemaphoreType.DMA,
            pltpu.SemaphoreType.DMA,
            pltpu.SemaphoreType.DMA,
        ],
    )
    def gkern(sq_hbm, sm_hbm, dq_hbm, dm_hbm, esrc_hbm, edst_hbm,
              sqo_hbm, smo_hbm, dqo_hbm, dmo_hbm,
              idx_s, idx_d, r_sq, r_sm, r_dq, r_dm,
              sem1, sem2, sem3, sem4):
        wid = lax.axis_index("s") * info.num_cores + lax.axis_index("c")
        base = wid * per_w

        def step(j, _):
            off = base + j * CH
            pltpu.sync_copy(esrc_hbm.at[pl.ds(off, CH)], idx_s)
            pltpu.sync_copy(edst_hbm.at[pl.ds(off, CH)], idx_d)
            c1 = pltpu.async_copy(sq_hbm.at[idx_s], r_sq, sem1)
            c3 = pltpu.async_copy(dq_hbm.at[idx_d], r_dq, sem3)
            c1.wait()
            c3.wait()
            c2 = pltpu.async_copy(sm_hbm.at[idx_s], r_sm, sem2)
            c4 = pltpu.async_copy(dm_hbm.at[idx_d], r_dm, sem4)
            pltpu.sync_copy(r_sq, sqo_hbm.at[pl.ds(off, CH)])
            pltpu.sync_copy(r_dq, dqo_hbm.at[pl.ds(off, CH)])
            c2.wait()
            c4.wait()
            pltpu.sync_copy(r_sm, smo_hbm.at[pl.ds(off, CH)])
            pltpu.sync_copy(r_dm, dmo_hbm.at[pl.ds(off, CH)])
            return ()

        lax.fori_loop(0, n_ch, step, ())

    return gkern(sq_tab, sm_tab, dq_tab, dm_tab, esrc, edst)


def _sc_scatter(msg4, edst, n_nodes, n_edges):
    info = plsc.get_sparse_core_info()
    ns = info.num_subcores
    per_s = n_edges // ns
    CH = 80
    n_ch = per_s // CH
    n_pad = ((n_nodes + 8 * ns - 1) // (8 * ns)) * (8 * ns)
    rows_per_s = n_pad // ns
    ZCH = 128
    mesh = plsc.VectorSubcoreMesh(core_axis_name="c", subcore_axis_name="s")

    @functools.partial(
        pl.kernel, mesh=mesh,
        out_type=jax.ShapeDtypeStruct((N_HEADS, n_pad, MSG_W), F32),
        scratch_types=[
            pltpu.VMEM((CH, MSG_W), F32),
            pltpu.VMEM((CH,), jnp.int32),
            pltpu.VMEM((ZCH, MSG_W), F32),
            pltpu.VMEM_SHARED((n_pad, MSG_W), F32),
        ],
    )
    def skern(msg4_hbm, edst_hbm, out_hbm, buf, idxb, zbuf, acc):
        core = lax.axis_index("c")
        sid = lax.axis_index("s")

        zbuf[...] = jnp.zeros((ZCH, MSG_W), F32)

        def one_head(h):
            for zz in range(rows_per_s // ZCH):
                pltpu.sync_copy(
                    zbuf, acc.at[pl.ds(sid * rows_per_s + zz * ZCH, ZCH)])
            plsc.subcore_barrier()

            def step(j, _):
                off = sid * per_s + j * CH
                pltpu.sync_copy(edst_hbm.at[pl.ds(off, CH)], idxb)
                pltpu.sync_copy(msg4_hbm.at[h, pl.ds(off, CH)], buf)
                pltpu.sync_copy(buf, acc.at[idxb], add=True)
                return ()

            lax.fori_loop(0, n_ch, step, ())
            plsc.subcore_barrier()
            pltpu.sync_copy(
                acc.at[pl.ds(sid * rows_per_s, rows_per_s)],
                out_hbm.at[h, pl.ds(sid * rows_per_s, rows_per_s)])
            plsc.subcore_barrier()

        one_head(core * 2)
        one_head(core * 2 + 1)

    return skern(msg4, edst)


def kernel(node_atom, pos, edge_src, edge_dst, batch, params):
    n_nodes = node_atom.shape[0]
    n_edges = edge_src.shape[0]
    esrc = edge_src.astype(jnp.int32)
    edst = edge_dst.astype(jnp.int32)
    sq_tab, dq_tab, sm_tab, dm_tab, skip_n = _node_stage(
        node_atom, pos, params, n_nodes)
    n_sl = 5
    e_sl = n_edges // n_sl
    eouts, parts = [], []
    for i in range(n_sl):
        lo = i * e_sl
        sq_g, sm_g, dq_g, dm_g = _sc_gather(
            sq_tab, sm_tab, dq_tab, dm_tab,
            lax.dynamic_slice(esrc, (lo,), (e_sl,)),
            lax.dynamic_slice(edst, (lo,), (e_sl,)), e_sl)
        eo, msg4 = _edge_stage(sq_g, sm_g, dq_g, dm_g, params, e_sl)
        eouts.append(eo)
        parts.append(_sc_scatter(msg4,
                                 lax.dynamic_slice(edst, (lo,), (e_sl,)),
                                 n_nodes, e_sl))
    edge_out = jnp.concatenate(eouts, axis=0)
    node_out = _node_out_stage(parts, skip_n, params, n_nodes)
    return node_out, edge_out


# speedup vs baseline: 3.1427x; 1.0009x over previous
import functools

import jax
import jax.numpy as jnp
from jax import lax
from jax.experimental import pallas as pl
from jax.experimental.pallas import tpu as pltpu
from jax.experimental.pallas import tpu_sc as plsc

F32 = jnp.float32

TAB_W = 640
QK_W = 128
MSG_TAB_W = 512
MSG_PACK_W = 256
N_HEADS = 4
D_QK = 16
D_HEAD = 120
D_NODE = 480
D_EDGE = 184
N_BASIS = 128
D_SCALAR = 128

NODE_BLK = 1000
EDGE_BLK = 640
MSG_W = 128


def _ln(x, eps=1e-6):
    m = jnp.mean(x, axis=-1, keepdims=True)
    d = x - m
    v = jnp.mean(d * d, axis=-1, keepdims=True)
    return d / jnp.sqrt(v + eps)


def _silu(x):
    return x * jax.nn.sigmoid(x)


def _node_stage_body(natom_ref, pos_ref, atab_ref,
                     qW1_ref, qb1_ref, qg1_ref, qbe1_ref, qW2_ref, qb2_ref,
                     kW1_ref, kb1_ref, kg1_ref, kbe1_ref, kW2_ref, kb2_ref,
                     preWs_ref, preWd_ref, skipW_ref,
                     sq_ref, dq_ref, sm_ref, dm_ref, skip_ref):
    atom = natom_ref[0, 0, :]
    oh = (atom.reshape(NODE_BLK, 1) ==
          lax.broadcasted_iota(jnp.int32, (1, 128), 1)).astype(F32)
    node_fea = jnp.dot(oh, atab_ref[...], preferred_element_type=F32)
    nf = _ln(node_fea)
    ns = nf[:, :D_SCALAR]

    def mlp(W1, b1, g1, be1, W2, b2):
        h = jnp.dot(ns, W1[...], preferred_element_type=F32) + b1[...]
        h = _silu(_ln(h) * g1[...] + be1[...])
        return jnp.dot(h, W2[...], preferred_element_type=F32) + b2[...]

    q_node = mlp(qW1_ref, qb1_ref, qg1_ref, qbe1_ref, qW2_ref, qb2_ref)
    k_node = mlp(kW1_ref, kb1_ref, kg1_ref, kbe1_ref, kW2_ref, kb2_ref)
    msg_s = jnp.dot(nf, preWs_ref[...], preferred_element_type=F32)
    msg_d = jnp.dot(nf, preWd_ref[...], preferred_element_type=F32)
    pos = pos_ref[...]
    zpad = jnp.zeros((NODE_BLK, 48), F32)
    sq_ref[...] = jnp.concatenate([k_node, pos, zpad], axis=1)
    dq_ref[...] = jnp.concatenate([q_node, pos, zpad], axis=1)

    def pack_pair(m):
        lo = m[:, 0:256]
        hi = jnp.concatenate(
            [m[:, 256:480], jnp.zeros((NODE_BLK, 32), F32)], axis=1)
        ulo = ((lax.bitcast_convert_type(lo, jnp.int32) >> 16) &
               jnp.int32(0xFFFF))
        uhi = (lax.bitcast_convert_type(hi, jnp.int32) &
               jnp.int32(-65536))
        return ulo | uhi

    sm_ref[...] = pack_pair(msg_s)
    dm_ref[...] = pack_pair(msg_d)
    skip_ref[...] = jnp.dot(node_fea, skipW_ref[...],
                            preferred_element_type=F32)


def _node_stage(node_atom, pos, p, n_nodes, interpret=False):
    nblk = n_nodes // NODE_BLK
    natom3 = node_atom.astype(jnp.int32).reshape(nblk, 1, NODE_BLK)
    pos16 = jnp.pad(pos.astype(F32), ((0, 0), (0, 13)))

    def full(a):
        return pl.BlockSpec(a.shape, lambda i: (0,) * a.ndim)

    r1 = lambda v: v.reshape(1, -1)
    weights = [p['atom_table'],
               p['q_W1'], r1(p['q_b1']), r1(p['q_g1']), r1(p['q_be1']),
               p['q_W2'], r1(p['q_b2']),
               p['k_W1'], r1(p['k_b1']), r1(p['k_g1']), r1(p['k_be1']),
               p['k_W2'], r1(p['k_b2']),
               p['pre_Ws'], p['pre_Wd'], p['skip_n_W']]
    out = pl.pallas_call(
        _node_stage_body,
        grid=(nblk,),
        in_specs=[pl.BlockSpec((1, 1, NODE_BLK), lambda i: (i, 0, 0)),
                  pl.BlockSpec((NODE_BLK, 16), lambda i: (i, 0))] +
                 [full(w) for w in weights],
        out_specs=[pl.BlockSpec((NODE_BLK, QK_W), lambda i: (i, 0)),
                   pl.BlockSpec((NODE_BLK, QK_W), lambda i: (i, 0)),
                   pl.BlockSpec((NODE_BLK, MSG_PACK_W), lambda i: (i, 0)),
                   pl.BlockSpec((NODE_BLK, MSG_PACK_W), lambda i: (i, 0)),
                   pl.BlockSpec((NODE_BLK, D_NODE), lambda i: (i, 0))],
        out_shape=[jax.ShapeDtypeStruct((n_nodes, QK_W), F32),
                   jax.ShapeDtypeStruct((n_nodes, QK_W), F32),
                   jax.ShapeDtypeStruct((n_nodes, MSG_PACK_W), jnp.int32),
                   jax.ShapeDtypeStruct((n_nodes, MSG_PACK_W), jnp.int32),
                   jax.ShapeDtypeStruct((n_nodes, D_NODE), F32)],
        interpret=interpret,
    )(natom3, pos16, *weights)
    return out


def _edge_stage_body(sq_ref, sm_ref, dq_ref, dm_ref,
                     Wep_ref, bep_ref, shc_ref,
                     aW1_ref, ab1_ref, ag1_ref, abe1_ref,
                     aW2_ref, ab2_ref, ag2_ref, abe2_ref,
                     aW3_ref, ab3_ref,
                     preWe_ref, preb_ref,
                     c1W_ref, c1Wr_ref, c1b_ref,
                     c2W_ref, c2Wr_ref, c2b_ref,
                     onW_ref, lineW_ref, lineb_ref, skipeW_ref,
                     eout_ref, msg4_ref):
    s_qk = sq_ref[...]
    d_qk = dq_ref[...]
    B = EDGE_BLK

    ev = d_qk[:, 64:80] - s_qk[:, 64:80]
    ev2 = ev * ev
    r2_1 = jnp.sum(ev2, axis=1, keepdims=True)
    rb = jnp.sqrt(jnp.broadcast_to(r2_1, (B, N_BASIS)))
    invr = 1.0 / jnp.maximum(rb, 1e-12)
    X = jnp.broadcast_to(ev[:, 0:1], (B, N_BASIS)) * invr
    Y = jnp.broadcast_to(ev[:, 1:2], (B, N_BASIS)) * invr
    Z = jnp.broadcast_to(ev[:, 2:3], (B, N_BASIS)) * invr

    shc = shc_ref[...]
    C1, CX, CY, CZ = shc[0:1], shc[1:2], shc[2:3], shc[3:4]
    CXY, CYZ, CXZ = shc[4:5], shc[5:6], shc[6:7]
    CXX, CYY, CZZ = shc[7:8], shc[8:9], shc[9:10]
    sh128 = (C1 +
             X * (CX + CXY * Y + CXZ * Z + CXX * X) +
             Y * (CY + CYZ * Z + CYY * Y) +
             Z * (CZ + CZZ * Z))
    edge_fea = (jnp.dot(sh128, Wep_ref[...], preferred_element_type=F32) +
                bep_ref[...])
    ef = _ln(edge_fea)

    centers = (lax.broadcasted_iota(jnp.int32, (1, N_BASIS), 1).astype(F32)
               * (10.0 / (N_BASIS - 1)))
    width = 0.5 * 10.0 / N_BASIS
    dr = rb - centers
    rbf = jnp.exp(-(dr * dr) / (2.0 * width * width))

    a = jnp.dot(rbf, aW1_ref[...], preferred_element_type=F32) + ab1_ref[...]
    a = _silu(_ln(a) * ag1_ref[...] + abe1_ref[...])
    a = jnp.dot(a, aW2_ref[...], preferred_element_type=F32) + ab2_ref[...]
    a = _silu(_ln(a) * ag2_ref[...] + abe2_ref[...])
    edge_bias = jnp.dot(a, aW3_ref[...], preferred_element_type=F32) + ab3_ref[...]

    qk = d_qk[:, 0:64] * s_qk[:, 0:64]
    qk_h = jnp.concatenate(
        [jnp.sum(qk[:, h * D_QK:(h + 1) * D_QK], axis=1, keepdims=True)
         for h in range(N_HEADS)], axis=1)
    alpha = qk_h * (1.0 / (D_QK ** 0.5)) + edge_bias

    def unpack_pair(ref):
        u = lax.bitcast_convert_type(ref[...], jnp.int32)
        lo = lax.bitcast_convert_type(u << 16, F32)
        hi = lax.bitcast_convert_type(u & jnp.int32(-65536), F32)
        return jnp.concatenate([lo, hi[:, 0:D_NODE - MSG_PACK_W]], axis=1)

    msg = ((unpack_pair(sm_ref) + unpack_pair(dm_ref)) +
           jnp.dot(ef, preWe_ref[...], preferred_element_type=F32) +
           preb_ref[...])
    t = (jnp.dot(msg, c1W_ref[...], preferred_element_type=F32) +
         jnp.dot(rbf, c1Wr_ref[...], preferred_element_type=F32) + c1b_ref[...])
    v = _silu(_ln(t))
    v = (jnp.dot(v, c2W_ref[...], preferred_element_type=F32) +
         jnp.dot(rbf, c2Wr_ref[...], preferred_element_type=F32) + c2b_ref[...])
    von = jnp.dot(msg, onW_ref[...], preferred_element_type=F32)
    r_480 = jnp.sqrt(jnp.broadcast_to(r2_1, (B, D_NODE)))
    v = jnp.where(r_480 < 1e-10, von, v)

    em = jnp.concatenate(
        [v[:, h * D_HEAD:(h + 1) * D_HEAD] * alpha[:, h:h + 1]
         for h in range(N_HEADS)], axis=1)

    eout_ref[...] = (jnp.dot(em, lineW_ref[...], preferred_element_type=F32) +
                     lineb_ref[...] +
                     jnp.dot(edge_fea, skipeW_ref[...],
                             preferred_element_type=F32))
    zpad8 = jnp.zeros((B, MSG_W - D_HEAD), F32)
    for h in range(N_HEADS):
        msg4_ref[h, :, :] = jnp.concatenate(
            [em[:, h * D_HEAD:(h + 1) * D_HEAD], zpad8], axis=1)


def _edge_stage(s_qk, s_msg, d_qk, d_msg, p, n_edges, interpret=False):
    eblk = n_edges // EDGE_BLK

    def full(a):
        return pl.BlockSpec(a.shape, lambda i: (0,) * a.ndim)

    r1 = lambda v: v.reshape(1, -1)
    s3, s5, s15 = 3.0 ** 0.5, 5.0 ** 0.5, 15.0 ** 0.5
    shc = jnp.zeros((10, 16), F32)
    shc = shc.at[0, 0].set(1.0).at[0, 6].set(-s5 / 2.0)
    shc = shc.at[1, 1].set(s3).at[2, 2].set(s3).at[3, 3].set(s3)
    shc = shc.at[4, 4].set(s15).at[5, 5].set(s15).at[6, 7].set(s15)
    shc = shc.at[7, 8].set(s15 / 2.0).at[8, 8].set(-s15 / 2.0)
    shc = shc.at[9, 6].set(3.0 * s5 / 2.0)
    shc = jnp.pad(shc, ((0, 0), (0, N_BASIS - 16)))
    wep128 = jnp.pad(p['W_edge_pre'], ((0, N_BASIS - 9), (0, 0)))
    weights = [wep128, r1(p['b_edge_pre']), shc,
               p['a_W1'], r1(p['a_b1']), r1(p['a_g1']), r1(p['a_be1']),
               p['a_W2'], r1(p['a_b2']), r1(p['a_g2']), r1(p['a_be2']),
               p['a_W3'], r1(p['a_b3']),
               p['pre_We'], r1(p['pre_b']),
               p['c1_W'], p['c1_Wr'], r1(p['c1_b']),
               p['c2_W'], p['c2_Wr'], r1(p['c2_b']),
               p['on_W'], p['lin_e_W'], r1(p['lin_e_b']), p['skip_e_W']]
    return pl.pallas_call(
        _edge_stage_body,
        grid=(eblk,),
        in_specs=[pl.BlockSpec((EDGE_BLK, QK_W), lambda i: (i, 0)),
                  pl.BlockSpec((EDGE_BLK, MSG_PACK_W), lambda i: (i, 0)),
                  pl.BlockSpec((EDGE_BLK, QK_W), lambda i: (i, 0)),
                  pl.BlockSpec((EDGE_BLK, MSG_PACK_W), lambda i: (i, 0))] +
                 [full(w) for w in weights],
        out_specs=[pl.BlockSpec((EDGE_BLK, D_EDGE), lambda i: (i, 0)),
                   pl.BlockSpec((N_HEADS, EDGE_BLK, MSG_W),
                                lambda i: (0, i, 0))],
        out_shape=[jax.ShapeDtypeStruct((n_edges, D_EDGE), F32),
                   jax.ShapeDtypeStruct((N_HEADS, n_edges, MSG_W), F32)],
        interpret=interpret,
    )(s_qk, s_msg, d_qk, d_msg, *weights)


def _node_out_body(*refs):
    n_parts = len(refs) - 4
    part_refs = refs[:n_parts]
    skip_ref, linW_ref, linb_ref, out_ref = refs[n_parts:]
    m4 = part_refs[0][...]
    for pr in part_refs[1:]:
        m4 = m4 + pr[...]
    acc = skip_ref[...] + linb_ref[...]
    for h in range(N_HEADS):
        acc = acc + jnp.dot(m4[h], linW_ref[h, :, :],
                            preferred_element_type=F32)
    out_ref[...] = acc


def _node_out_stage(msg4n_parts, skip_n, p, n_nodes, interpret=False):
    nblk = n_nodes // NODE_BLK
    linW4 = jnp.pad(p['lin_W'].reshape(N_HEADS, D_HEAD, D_NODE),
                    ((0, 0), (0, MSG_W - D_HEAD), (0, 0)))
    linb = p['lin_b'].reshape(1, -1)
    part_spec = pl.BlockSpec((N_HEADS, NODE_BLK, MSG_W), lambda i: (0, i, 0))
    return pl.pallas_call(
        _node_out_body,
        grid=(nblk,),
        in_specs=[part_spec] * len(msg4n_parts) +
                 [pl.BlockSpec((NODE_BLK, D_NODE), lambda i: (i, 0)),
                  pl.BlockSpec(linW4.shape, lambda i: (0, 0, 0)),
                  pl.BlockSpec(linb.shape, lambda i: (0, 0))],
        out_specs=pl.BlockSpec((NODE_BLK, D_NODE), lambda i: (i, 0)),
        out_shape=jax.ShapeDtypeStruct((n_nodes, D_NODE), F32),
        interpret=interpret,
    )(*msg4n_parts, skip_n, linW4, linb)


def _sc_gather(sq_tab, sm_tab, dq_tab, dm_tab, esrc, edst, n_edges):
    info = plsc.get_sparse_core_info()
    nw = info.num_cores * info.num_subcores
    per_w = n_edges // nw
    CH = 40
    n_ch = per_w // CH
    MW = MSG_PACK_W
    mesh = plsc.VectorSubcoreMesh(core_axis_name="c", subcore_axis_name="s")

    @functools.partial(
        pl.kernel, mesh=mesh,
        out_type=[jax.ShapeDtypeStruct((n_edges, QK_W), F32),
                  jax.ShapeDtypeStruct((n_edges, MW), jnp.int32),
                  jax.ShapeDtypeStruct((n_edges, QK_W), F32),
                  jax.ShapeDtypeStruct((n_edges, MW), jnp.int32)],
        scratch_types=[
            pltpu.VMEM((CH,), jnp.int32),
            pltpu.VMEM((CH,), jnp.int32),
            pltpu.VMEM((CH, QK_W), F32),
            pltpu.VMEM((CH, MW), jnp.int32),
            pltpu.VMEM((CH, QK_W), F32),
            pltpu.VMEM((CH, MW), jnp.int32),
            pltpu.SemaphoreType.DMA,
            pltpu.SemaphoreType.DMA,
            pltpu.SemaphoreType.DMA,
            pltpu.SemaphoreType.DMA,
        ],
    )
    def gkern(sq_hbm, sm_hbm, dq_hbm, dm_hbm, esrc_hbm, edst_hbm,
              sqo_hbm, smo_hbm, dqo_hbm, dmo_hbm,
              idx_s, idx_d, r_sq, r_sm, r_dq, r_dm,
              sem1, sem2, sem3, sem4):
        wid = lax.axis_index("s") * info.num_cores + lax.axis_index("c")
        base = wid * per_w

        def step(j, _):
            off = base + j * CH
            pltpu.sync_copy(esrc_hbm.at[pl.ds(off, CH)], idx_s)
            pltpu.sync_copy(edst_hbm.at[pl.ds(off, CH)], idx_d)
            c1 = pltpu.async_copy(sq_hbm.at[idx_s], r_sq, sem1)
            c3 = pltpu.async_copy(dq_hbm.at[idx_d], r_dq, sem3)
            c1.wait()
            c3.wait()
            c2 = pltpu.async_copy(sm_hbm.at[idx_s], r_sm, sem2)
            c4 = pltpu.async_copy(dm_hbm.at[idx_d], r_dm, sem4)
            pltpu.sync_copy(r_sq, sqo_hbm.at[pl.ds(off, CH)])
            pltpu.sync_copy(r_dq, dqo_hbm.at[pl.ds(off, CH)])
            c2.wait()
            c4.wait()
            pltpu.sync_copy(r_sm, smo_hbm.at[pl.ds(off, CH)])
            pltpu.sync_copy(r_dm, dmo_hbm.at[pl.ds(off, CH)])
            return ()

        lax.fori_loop(0, n_ch, step, ())

    return gkern(sq_tab, sm_tab, dq_tab, dm_tab, esrc, edst)


def _sc_scatter(msg4, edst, n_nodes, n_edges):
    info = plsc.get_sparse_core_info()
    ns = info.num_subcores
    per_s = n_edges // ns
    CH = 80
    n_ch = per_s // CH
    n_pad = ((n_nodes + 8 * ns - 1) // (8 * ns)) * (8 * ns)
    rows_per_s = n_pad // ns
    ZCH = 128
    mesh = plsc.VectorSubcoreMesh(core_axis_name="c", subcore_axis_name="s")

    @functools.partial(
        pl.kernel, mesh=mesh,
        out_type=jax.ShapeDtypeStruct((N_HEADS, n_pad, MSG_W), F32),
        scratch_types=[
            pltpu.VMEM((CH, MSG_W), F32),
            pltpu.VMEM((CH,), jnp.int32),
            pltpu.VMEM((ZCH, MSG_W), F32),
            pltpu.VMEM_SHARED((n_pad, MSG_W), F32),
        ],
    )
    def skern(msg4_hbm, edst_hbm, out_hbm, buf, idxb, zbuf, acc):
        core = lax.axis_index("c")
        sid = lax.axis_index("s")

        zbuf[...] = jnp.zeros((ZCH, MSG_W), F32)

        def one_head(h):
            for zz in range(rows_per_s // ZCH):
                pltpu.sync_copy(
                    zbuf, acc.at[pl.ds(sid * rows_per_s + zz * ZCH, ZCH)])
            plsc.subcore_barrier()

            def step(j, _):
                off = sid * per_s + j * CH
                pltpu.sync_copy(edst_hbm.at[pl.ds(off, CH)], idxb)
                pltpu.sync_copy(msg4_hbm.at[h, pl.ds(off, CH)], buf)
                pltpu.sync_copy(buf, acc.at[idxb], add=True)
                return ()

            lax.fori_loop(0, n_ch, step, ())
            plsc.subcore_barrier()
            pltpu.sync_copy(
                acc.at[pl.ds(sid * rows_per_s, rows_per_s)],
                out_hbm.at[h, pl.ds(sid * rows_per_s, rows_per_s)])
            plsc.subcore_barrier()

        one_head(core * 2)
        one_head(core * 2 + 1)

    return skern(msg4, edst)


def kernel(node_atom, pos, edge_src, edge_dst, batch, params):
    n_nodes = node_atom.shape[0]
    n_edges = edge_src.shape[0]
    esrc = edge_src.astype(jnp.int32)
    edst = edge_dst.astype(jnp.int32)
    sq_tab, dq_tab, sm_tab, dm_tab, skip_n = _node_stage(
        node_atom, pos, params, n_nodes)
    n_sl = 5
    e_sl = n_edges // n_sl
    eouts, parts = [], []
    for i in range(n_sl):
        lo = i * e_sl
        sq_g, sm_g, dq_g, dm_g = _sc_gather(
            sq_tab, sm_tab, dq_tab, dm_tab,
            lax.dynamic_slice(esrc, (lo,), (e_sl,)),
            lax.dynamic_slice(edst, (lo,), (e_sl,)), e_sl)
        eo, msg4 = _edge_stage(sq_g, sm_g, dq_g, dm_g, params, e_sl)
        eouts.append(eo)
        parts.append(_sc_scatter(msg4,
                                 lax.dynamic_slice(edst, (lo,), (e_sl,)),
                                 n_nodes, e_sl))
    edge_out = jnp.concatenate(eouts, axis=0)
    node_out = _node_out_stage(parts, skip_n, params, n_nodes)
    return node_out, edge_out


# speedup vs baseline: 3.2140x; 1.0227x over previous
import functools

import jax
import jax.numpy as jnp
from jax import lax
from jax.experimental import pallas as pl
from jax.experimental.pallas import tpu as pltpu
from jax.experimental.pallas import tpu_sc as plsc

F32 = jnp.float32

TAB_W = 640
QK_W = 128
MSG_TAB_W = 512
MSG_PACK_W = 256
N_HEADS = 4
D_QK = 16
D_HEAD = 120
D_NODE = 480
D_EDGE = 184
N_BASIS = 128
D_SCALAR = 128

NODE_BLK = 1000
EDGE_BLK = 640
MSG_W = 128


def _ln(x, eps=1e-6):
    m = jnp.mean(x, axis=-1, keepdims=True)
    d = x - m
    v = jnp.mean(d * d, axis=-1, keepdims=True)
    return d / jnp.sqrt(v + eps)


def _silu(x):
    return x * jax.nn.sigmoid(x)


def _node_stage_body(natom_ref, pos_ref, atab_ref,
                     qW1_ref, qb1_ref, qg1_ref, qbe1_ref, qW2_ref, qb2_ref,
                     kW1_ref, kb1_ref, kg1_ref, kbe1_ref, kW2_ref, kb2_ref,
                     preWs_ref, preWd_ref, skipW_ref,
                     sq_ref, dq_ref, sm_ref, dm_ref, skip_ref):
    atom = natom_ref[0, 0, :]
    oh = (atom.reshape(NODE_BLK, 1) ==
          lax.broadcasted_iota(jnp.int32, (1, 128), 1)).astype(F32)
    node_fea = jnp.dot(oh, atab_ref[...], preferred_element_type=F32)
    nf = _ln(node_fea)
    ns = nf[:, :D_SCALAR]

    def mlp(W1, b1, g1, be1, W2, b2):
        h = jnp.dot(ns, W1[...], preferred_element_type=F32) + b1[...]
        h = _silu(_ln(h) * g1[...] + be1[...])
        return jnp.dot(h, W2[...], preferred_element_type=F32) + b2[...]

    q_node = mlp(qW1_ref, qb1_ref, qg1_ref, qbe1_ref, qW2_ref, qb2_ref)
    k_node = mlp(kW1_ref, kb1_ref, kg1_ref, kbe1_ref, kW2_ref, kb2_ref)
    msg_s = jnp.dot(nf, preWs_ref[...], preferred_element_type=F32)
    msg_d = jnp.dot(nf, preWd_ref[...], preferred_element_type=F32)
    pos = pos_ref[...]
    zpad = jnp.zeros((NODE_BLK, 48), F32)
    sq_ref[...] = jnp.concatenate([k_node, pos, zpad], axis=1)
    dq_ref[...] = jnp.concatenate([q_node, pos, zpad], axis=1)

    def pack_pair(m):
        lo = m[:, 0:256]
        hi = jnp.concatenate(
            [m[:, 256:480], jnp.zeros((NODE_BLK, 32), F32)], axis=1)
        ulo = ((lax.bitcast_convert_type(lo, jnp.int32) >> 16) &
               jnp.int32(0xFFFF))
        uhi = (lax.bitcast_convert_type(hi, jnp.int32) &
               jnp.int32(-65536))
        return ulo | uhi

    sm_ref[...] = pack_pair(msg_s)
    dm_ref[...] = pack_pair(msg_d)
    skip_ref[...] = jnp.dot(node_fea, skipW_ref[...],
                            preferred_element_type=F32)


def _node_stage(node_atom, pos, p, n_nodes, interpret=False):
    nblk = n_nodes // NODE_BLK
    natom3 = node_atom.astype(jnp.int32).reshape(nblk, 1, NODE_BLK)
    pos16 = jnp.pad(pos.astype(F32), ((0, 0), (0, 13)))

    def full(a):
        return pl.BlockSpec(a.shape, lambda i: (0,) * a.ndim)

    r1 = lambda v: v.reshape(1, -1)
    weights = [p['atom_table'],
               p['q_W1'], r1(p['q_b1']), r1(p['q_g1']), r1(p['q_be1']),
               p['q_W2'], r1(p['q_b2']),
               p['k_W1'], r1(p['k_b1']), r1(p['k_g1']), r1(p['k_be1']),
               p['k_W2'], r1(p['k_b2']),
               p['pre_Ws'], p['pre_Wd'], p['skip_n_W']]
    out = pl.pallas_call(
        _node_stage_body,
        grid=(nblk,),
        in_specs=[pl.BlockSpec((1, 1, NODE_BLK), lambda i: (i, 0, 0)),
                  pl.BlockSpec((NODE_BLK, 16), lambda i: (i, 0))] +
                 [full(w) for w in weights],
        out_specs=[pl.BlockSpec((NODE_BLK, QK_W), lambda i: (i, 0)),
                   pl.BlockSpec((NODE_BLK, QK_W), lambda i: (i, 0)),
                   pl.BlockSpec((NODE_BLK, MSG_PACK_W), lambda i: (i, 0)),
                   pl.BlockSpec((NODE_BLK, MSG_PACK_W), lambda i: (i, 0)),
                   pl.BlockSpec((NODE_BLK, D_NODE), lambda i: (i, 0))],
        out_shape=[jax.ShapeDtypeStruct((n_nodes, QK_W), F32),
                   jax.ShapeDtypeStruct((n_nodes, QK_W), F32),
                   jax.ShapeDtypeStruct((n_nodes, MSG_PACK_W), jnp.int32),
                   jax.ShapeDtypeStruct((n_nodes, MSG_PACK_W), jnp.int32),
                   jax.ShapeDtypeStruct((n_nodes, D_NODE), F32)],
        interpret=interpret,
    )(natom3, pos16, *weights)
    return out


def _edge_stage_body(sq_ref, sm_ref, dq_ref, dm_ref,
                     Wep_ref, bep_ref, shc_ref,
                     aW1_ref, ab1_ref, ag1_ref, abe1_ref,
                     aW2_ref, ab2_ref, ag2_ref, abe2_ref,
                     aW3_ref, ab3_ref,
                     preWe_ref, preb_ref,
                     c1W_ref, c1Wr_ref, c1b_ref,
                     c2W_ref, c2Wr_ref, c2b_ref,
                     onW_ref, lineW_ref, lineb_ref, skipeW_ref,
                     eout_ref, msg4_ref):
    s_qk = sq_ref[...]
    d_qk = dq_ref[...]
    B = EDGE_BLK

    ev = d_qk[:, 64:80] - s_qk[:, 64:80]
    ev2 = ev * ev
    r2_1 = jnp.sum(ev2, axis=1, keepdims=True)
    rb = jnp.sqrt(jnp.broadcast_to(r2_1, (B, N_BASIS)))
    invr = 1.0 / jnp.maximum(rb, 1e-12)
    X = jnp.broadcast_to(ev[:, 0:1], (B, N_BASIS)) * invr
    Y = jnp.broadcast_to(ev[:, 1:2], (B, N_BASIS)) * invr
    Z = jnp.broadcast_to(ev[:, 2:3], (B, N_BASIS)) * invr

    shc = shc_ref[...]
    C1, CX, CY, CZ = shc[0:1], shc[1:2], shc[2:3], shc[3:4]
    CXY, CYZ, CXZ = shc[4:5], shc[5:6], shc[6:7]
    CXX, CYY, CZZ = shc[7:8], shc[8:9], shc[9:10]
    sh128 = (C1 +
             X * (CX + CXY * Y + CXZ * Z + CXX * X) +
             Y * (CY + CYZ * Z + CYY * Y) +
             Z * (CZ + CZZ * Z))
    edge_fea = (jnp.dot(sh128, Wep_ref[...], preferred_element_type=F32) +
                bep_ref[...])
    ef = _ln(edge_fea)

    centers = (lax.broadcasted_iota(jnp.int32, (1, N_BASIS), 1).astype(F32)
               * (10.0 / (N_BASIS - 1)))
    width = 0.5 * 10.0 / N_BASIS
    dr = rb - centers
    rbf = jnp.exp(-(dr * dr) / (2.0 * width * width))

    a = jnp.dot(rbf, aW1_ref[...], preferred_element_type=F32) + ab1_ref[...]
    a = _silu(_ln(a) * ag1_ref[...] + abe1_ref[...])
    a = jnp.dot(a, aW2_ref[...], preferred_element_type=F32) + ab2_ref[...]
    a = _silu(_ln(a) * ag2_ref[...] + abe2_ref[...])
    edge_bias = jnp.dot(a, aW3_ref[...], preferred_element_type=F32) + ab3_ref[...]

    qk = d_qk[:, 0:64] * s_qk[:, 0:64]
    qk_h = jnp.concatenate(
        [jnp.sum(qk[:, h * D_QK:(h + 1) * D_QK], axis=1, keepdims=True)
         for h in range(N_HEADS)], axis=1)
    alpha = qk_h * (1.0 / (D_QK ** 0.5)) + edge_bias

    def unpack_pair(ref):
        u = lax.bitcast_convert_type(ref[...], jnp.int32)
        lo = lax.bitcast_convert_type(u << 16, F32)
        hi = lax.bitcast_convert_type(u & jnp.int32(-65536), F32)
        return jnp.concatenate([lo, hi[:, 0:D_NODE - MSG_PACK_W]], axis=1)

    msg = ((unpack_pair(sm_ref) + unpack_pair(dm_ref)) +
           jnp.dot(ef, preWe_ref[...], preferred_element_type=F32) +
           preb_ref[...])
    t = (jnp.dot(msg, c1W_ref[...], preferred_element_type=F32) +
         jnp.dot(rbf, c1Wr_ref[...], preferred_element_type=F32) + c1b_ref[...])
    v = _silu(_ln(t))
    v = (jnp.dot(v, c2W_ref[...], preferred_element_type=F32) +
         jnp.dot(rbf, c2Wr_ref[...], preferred_element_type=F32) + c2b_ref[...])
    von = jnp.dot(msg, onW_ref[...], preferred_element_type=F32)
    r_480 = jnp.sqrt(jnp.broadcast_to(r2_1, (B, D_NODE)))
    v = jnp.where(r_480 < 1e-10, von, v)

    em = jnp.concatenate(
        [v[:, h * D_HEAD:(h + 1) * D_HEAD] * alpha[:, h:h + 1]
         for h in range(N_HEADS)], axis=1)

    eout_ref[...] = (jnp.dot(em, lineW_ref[...], preferred_element_type=F32) +
                     lineb_ref[...] +
                     jnp.dot(edge_fea, skipeW_ref[...],
                             preferred_element_type=F32))
    zpad8 = jnp.zeros((B, MSG_W - D_HEAD), F32)
    for h in range(N_HEADS):
        msg4_ref[h, :, :] = jnp.concatenate(
            [em[:, h * D_HEAD:(h + 1) * D_HEAD], zpad8], axis=1)


def _edge_stage(s_qk, s_msg, d_qk, d_msg, p, n_edges, interpret=False):
    eblk = n_edges // EDGE_BLK

    def full(a):
        return pl.BlockSpec(a.shape, lambda i: (0,) * a.ndim)

    r1 = lambda v: v.reshape(1, -1)
    s3, s5, s15 = 3.0 ** 0.5, 5.0 ** 0.5, 15.0 ** 0.5
    shc = jnp.zeros((10, 16), F32)
    shc = shc.at[0, 0].set(1.0).at[0, 6].set(-s5 / 2.0)
    shc = shc.at[1, 1].set(s3).at[2, 2].set(s3).at[3, 3].set(s3)
    shc = shc.at[4, 4].set(s15).at[5, 5].set(s15).at[6, 7].set(s15)
    shc = shc.at[7, 8].set(s15 / 2.0).at[8, 8].set(-s15 / 2.0)
    shc = shc.at[9, 6].set(3.0 * s5 / 2.0)
    shc = jnp.pad(shc, ((0, 0), (0, N_BASIS - 16)))
    wep128 = jnp.pad(p['W_edge_pre'], ((0, N_BASIS - 9), (0, 0)))
    weights = [wep128, r1(p['b_edge_pre']), shc,
               p['a_W1'], r1(p['a_b1']), r1(p['a_g1']), r1(p['a_be1']),
               p['a_W2'], r1(p['a_b2']), r1(p['a_g2']), r1(p['a_be2']),
               p['a_W3'], r1(p['a_b3']),
               p['pre_We'], r1(p['pre_b']),
               p['c1_W'], p['c1_Wr'], r1(p['c1_b']),
               p['c2_W'], p['c2_Wr'], r1(p['c2_b']),
               p['on_W'], p['lin_e_W'], r1(p['lin_e_b']), p['skip_e_W']]
    return pl.pallas_call(
        _edge_stage_body,
        grid=(eblk,),
        in_specs=[pl.BlockSpec((EDGE_BLK, QK_W), lambda i: (i, 0)),
                  pl.BlockSpec((EDGE_BLK, MSG_PACK_W), lambda i: (i, 0)),
                  pl.BlockSpec((EDGE_BLK, QK_W), lambda i: (i, 0)),
                  pl.BlockSpec((EDGE_BLK, MSG_PACK_W), lambda i: (i, 0))] +
                 [full(w) for w in weights],
        out_specs=[pl.BlockSpec((EDGE_BLK, D_EDGE), lambda i: (i, 0)),
                   pl.BlockSpec((N_HEADS, EDGE_BLK, MSG_W),
                                lambda i: (0, i, 0))],
        out_shape=[jax.ShapeDtypeStruct((n_edges, D_EDGE), F32),
                   jax.ShapeDtypeStruct((N_HEADS, n_edges, MSG_W), F32)],
        interpret=interpret,
    )(s_qk, s_msg, d_qk, d_msg, *weights)


def _node_out_body(*refs):
    n_parts = len(refs) - 4
    part_refs = refs[:n_parts]
    skip_ref, linW_ref, linb_ref, out_ref = refs[n_parts:]
    m4 = part_refs[0][...]
    for pr in part_refs[1:]:
        m4 = m4 + pr[...]
    acc = skip_ref[...] + linb_ref[...]
    for h in range(N_HEADS):
        acc = acc + jnp.dot(m4[h], linW_ref[h, :, :],
                            preferred_element_type=F32)
    out_ref[...] = acc


def _node_out_stage(msg4n_parts, skip_n, p, n_nodes, interpret=False):
    nblk = n_nodes // NODE_BLK
    linW4 = jnp.pad(p['lin_W'].reshape(N_HEADS, D_HEAD, D_NODE),
                    ((0, 0), (0, MSG_W - D_HEAD), (0, 0)))
    linb = p['lin_b'].reshape(1, -1)
    part_spec = pl.BlockSpec((N_HEADS, NODE_BLK, MSG_W), lambda i: (0, i, 0))
    return pl.pallas_call(
        _node_out_body,
        grid=(nblk,),
        in_specs=[part_spec] * len(msg4n_parts) +
                 [pl.BlockSpec((NODE_BLK, D_NODE), lambda i: (i, 0)),
                  pl.BlockSpec(linW4.shape, lambda i: (0, 0, 0)),
                  pl.BlockSpec(linb.shape, lambda i: (0, 0))],
        out_specs=pl.BlockSpec((NODE_BLK, D_NODE), lambda i: (i, 0)),
        out_shape=jax.ShapeDtypeStruct((n_nodes, D_NODE), F32),
        interpret=interpret,
    )(*msg4n_parts, skip_n, linW4, linb)


def _sc_gather(sq_tab, sm_tab, dq_tab, dm_tab, esrc, edst, n_edges):
    info = plsc.get_sparse_core_info()
    nw = info.num_cores * info.num_subcores
    per_w = n_edges // nw
    CH = 40
    n_ch = per_w // CH
    MW = MSG_PACK_W
    mesh = plsc.VectorSubcoreMesh(core_axis_name="c", subcore_axis_name="s")

    @functools.partial(
        pl.kernel, mesh=mesh,
        out_type=[jax.ShapeDtypeStruct((n_edges, QK_W), F32),
                  jax.ShapeDtypeStruct((n_edges, MW), jnp.int32),
                  jax.ShapeDtypeStruct((n_edges, QK_W), F32),
                  jax.ShapeDtypeStruct((n_edges, MW), jnp.int32)],
        scratch_types=[
            pltpu.VMEM((CH,), jnp.int32),
            pltpu.VMEM((CH,), jnp.int32),
            pltpu.VMEM((CH, QK_W), F32),
            pltpu.VMEM((CH, MW), jnp.int32),
            pltpu.VMEM((CH, QK_W), F32),
            pltpu.VMEM((CH, MW), jnp.int32),
            pltpu.SemaphoreType.DMA,
            pltpu.SemaphoreType.DMA,
            pltpu.SemaphoreType.DMA,
            pltpu.SemaphoreType.DMA,
        ],
    )
    def gkern(sq_hbm, sm_hbm, dq_hbm, dm_hbm, esrc_hbm, edst_hbm,
              sqo_hbm, smo_hbm, dqo_hbm, dmo_hbm,
              idx_s, idx_d, r_sq, r_sm, r_dq, r_dm,
              sem1, sem2, sem3, sem4):
        wid = lax.axis_index("s") * info.num_cores + lax.axis_index("c")
        base = wid * per_w

        def step(j, _):
            off = base + j * CH
            pltpu.sync_copy(esrc_hbm.at[pl.ds(off, CH)], idx_s)
            pltpu.sync_copy(edst_hbm.at[pl.ds(off, CH)], idx_d)
            c1 = pltpu.async_copy(sq_hbm.at[idx_s], r_sq, sem1)
            c3 = pltpu.async_copy(dq_hbm.at[idx_d], r_dq, sem3)
            c1.wait()
            c3.wait()
            c2 = pltpu.async_copy(sm_hbm.at[idx_s], r_sm, sem2)
            c4 = pltpu.async_copy(dm_hbm.at[idx_d], r_dm, sem4)
            pltpu.sync_copy(r_sq, sqo_hbm.at[pl.ds(off, CH)])
            pltpu.sync_copy(r_dq, dqo_hbm.at[pl.ds(off, CH)])
            c2.wait()
            c4.wait()
            pltpu.sync_copy(r_sm, smo_hbm.at[pl.ds(off, CH)])
            pltpu.sync_copy(r_dm, dmo_hbm.at[pl.ds(off, CH)])
            return ()

        lax.fori_loop(0, n_ch, step, ())

    return gkern(sq_tab, sm_tab, dq_tab, dm_tab, esrc, edst)


def _sc_scatter(msg4, edst, n_nodes, n_edges):
    info = plsc.get_sparse_core_info()
    ns = info.num_subcores
    per_s = n_edges // ns
    CH = 80
    n_ch = per_s // CH
    n_pad = ((n_nodes + 8 * ns - 1) // (8 * ns)) * (8 * ns)
    rows_per_s = n_pad // ns
    ZCH = 128
    mesh = plsc.VectorSubcoreMesh(core_axis_name="c", subcore_axis_name="s")

    @functools.partial(
        pl.kernel, mesh=mesh,
        out_type=jax.ShapeDtypeStruct((N_HEADS, n_pad, MSG_W), F32),
        scratch_types=[
            pltpu.VMEM((CH, MSG_W), F32),
            pltpu.VMEM((CH, MSG_W), F32),
            pltpu.VMEM((CH,), jnp.int32),
            pltpu.VMEM((CH,), jnp.int32),
            pltpu.VMEM((ZCH, MSG_W), F32),
            pltpu.VMEM_SHARED((n_pad, MSG_W), F32),
            pltpu.SemaphoreType.DMA,
            pltpu.SemaphoreType.DMA,
            pltpu.SemaphoreType.DMA,
            pltpu.SemaphoreType.DMA,
        ],
    )
    def skern(msg4_hbm, edst_hbm, out_hbm, buf0, buf1, idx0, idx1, zbuf,
              acc, semi0, semr0, semi1, semr1):
        core = lax.axis_index("c")
        sid = lax.axis_index("s")

        zbuf[...] = jnp.zeros((ZCH, MSG_W), F32)

        def one_head(h):
            for zz in range(rows_per_s // ZCH):
                pltpu.sync_copy(
                    zbuf, acc.at[pl.ds(sid * rows_per_s + zz * ZCH, ZCH)])
            plsc.subcore_barrier()

            ebase = sid * per_s

            def fetch(j, idxb, buf, semi, semr):
                off = ebase + j * CH
                pltpu.async_copy(edst_hbm.at[pl.ds(off, CH)], idxb, semi)
                pltpu.async_copy(msg4_hbm.at[h, pl.ds(off, CH)], buf, semr)

            def drain_add(idxb, buf, semi, semr):
                pltpu.make_async_copy(
                    edst_hbm.at[pl.ds(0, CH)], idxb, semi).wait()
                pltpu.make_async_copy(
                    msg4_hbm.at[h, pl.ds(0, CH)], buf, semr).wait()
                pltpu.sync_copy(buf, acc.at[idxb], add=True)

            fetch(0, idx0, buf0, semi0, semr0)

            def pair(jj, _):
                j0 = 2 * jj
                fetch(j0 + 1, idx1, buf1, semi1, semr1)
                drain_add(idx0, buf0, semi0, semr0)
                fetch(j0 + 2, idx0, buf0, semi0, semr0)
                drain_add(idx1, buf1, semi1, semr1)
                return ()

            lax.fori_loop(0, (n_ch - 1) // 2, pair, ())
            drain_add(idx0, buf0, semi0, semr0)
            plsc.subcore_barrier()
            pltpu.sync_copy(
                acc.at[pl.ds(sid * rows_per_s, rows_per_s)],
                out_hbm.at[h, pl.ds(sid * rows_per_s, rows_per_s)])
            plsc.subcore_barrier()

        one_head(core * 2)
        one_head(core * 2 + 1)

    return skern(msg4, edst)


def kernel(node_atom, pos, edge_src, edge_dst, batch, params):
    n_nodes = node_atom.shape[0]
    n_edges = edge_src.shape[0]
    esrc = edge_src.astype(jnp.int32)
    edst = edge_dst.astype(jnp.int32)
    sq_tab, dq_tab, sm_tab, dm_tab, skip_n = _node_stage(
        node_atom, pos, params, n_nodes)
    n_sl = 5
    e_sl = n_edges // n_sl
    eouts, parts = [], []
    for i in range(n_sl):
        lo = i * e_sl
        sq_g, sm_g, dq_g, dm_g = _sc_gather(
            sq_tab, sm_tab, dq_tab, dm_tab,
            lax.dynamic_slice(esrc, (lo,), (e_sl,)),
            lax.dynamic_slice(edst, (lo,), (e_sl,)), e_sl)
        eo, msg4 = _edge_stage(sq_g, sm_g, dq_g, dm_g, params, e_sl)
        eouts.append(eo)
        parts.append(_sc_scatter(msg4,
                                 lax.dynamic_slice(edst, (lo,), (e_sl,)),
                                 n_nodes, e_sl))
    edge_out = jnp.concatenate(eouts, axis=0)
    node_out = _node_out_stage(parts, skip_n, params, n_nodes)
    return node_out, edge_out
